# Initial kernel scaffold; baseline (speedup 1.0000x reference)
import jax
import jax.numpy as jnp
from jax.experimental import pallas as pl


def kernel(node_feats, edge_index, edge_feats, W, b):
    raise NotImplementedError("write your pallas kernel here")


# speedup vs baseline: 2.4309x; 2.4309x over previous
import jax
import jax.numpy as jnp
from jax import lax
from jax.experimental import pallas as pl
from jax.experimental.pallas import tpu as pltpu
from jax.experimental.pallas import tpu_sc as plsc

N = 10000
E = 320000
D = 128
NC = 2
NS = 16
L = 16
NH = N // NC
ACC = 5200

CH = 80
EPS = E // NS
NCHUNK = EPS // CH
CHKR = 200
NZCHK = ACC // CHKR
NWCHK = NH // CHKR


def _sc_body(x_hbm, src_hbm, dst_hbm, ef_hbm, out_hbm,
             acc_s, src_v, dst_v, ef_v, rows_v, stage_v, sem):
    c = lax.axis_index("c")
    s = lax.axis_index("s")

    def _zero_row(i, carry):
        for k in range(D // L):
            stage_v[i, pl.ds(k * L, L)] = jnp.zeros((L,), jnp.float32)
        return carry
    lax.fori_loop(0, CHKR, _zero_row, None)
    for k in range(2):
        t = s + k * NS

        @pl.when(t < NZCHK)
        def _():
            pltpu.sync_copy(stage_v, acc_s.at[pl.ds(t * CHKR, CHKR)])
    plsc.subcore_barrier()

    e0 = s * EPS
    dump = NH + s

    def _chunk(j, carry):
        eb = e0 + j * CH
        pltpu.sync_copy(src_hbm.at[pl.ds(eb, CH)], src_v)
        pltpu.sync_copy(dst_hbm.at[pl.ds(eb, CH)], dst_v)
        pltpu.sync_copy(ef_hbm.at[pl.ds(eb, CH)], ef_v)
        pltpu.async_copy(x_hbm.at[src_v], rows_v, sem).wait()

        for q in range(CH // L):
            d16 = dst_v[pl.ds(q * L, L)] - c * NH
            inr = (d16 >= 0) & (d16 < NH)
            dst_v[pl.ds(q * L, L)] = jnp.where(inr, d16, dump)

        gdn = lax.GatherDimensionNumbers(
            offset_dims=(), collapsed_slice_dims=(0,), start_index_map=(0,))
        for g in range(CH // L):
            efg = ef_v[pl.ds(g * L, L)]
            for jl in range(L):
                efb = lax.gather(
                    efg, jnp.full((L, 1), jl, jnp.int32), gdn,
                    slice_sizes=(1,),
                    mode=lax.GatherScatterMode.PROMISE_IN_BOUNDS)
                jj = g * L + jl
                for k in range(D // L):
                    rows_v[jj, pl.ds(k * L, L)] = (
                        rows_v[jj, pl.ds(k * L, L)] * efb)

        pltpu.sync_copy(rows_v, acc_s.at[dst_v], add=True)
        return carry
    lax.fori_loop(0, NCHUNK, _chunk, None)
    plsc.subcore_barrier()

    for k in range(2):
        t = s + k * NS

        @pl.when(t < NWCHK)
        def _():
            row = t * CHKR
            pltpu.sync_copy(acc_s.at[pl.ds(row, CHKR)],
                            out_hbm.at[pl.ds(c * NH + row, CHKR)])


@jax.jit
def _sc_segment_sum(x, src, dst, ef):
    mesh = plsc.VectorSubcoreMesh(core_axis_name="c", subcore_axis_name="s")
    return pl.kernel(
        _sc_body,
        out_type=jax.ShapeDtypeStruct((N, D), jnp.float32),
        mesh=mesh,
        scratch_types=[
            pltpu.VMEM_SHARED((ACC, D), jnp.float32),
            pltpu.VMEM((CH,), jnp.int32),
            pltpu.VMEM((CH,), jnp.int32),
            pltpu.VMEM((CH,), jnp.float32),
            pltpu.VMEM((CH, D), jnp.float32),
            pltpu.VMEM((CHKR, D), jnp.float32),
            pltpu.SemaphoreType.DMA,
        ],
    )(x, src, dst, ef)


MT = 2000


def _mm_body(x_ref, r_ref, w_ref, b_ref, o_ref):
    dn = (((1,), (1,)), ((), ()))
    o_ref[...] = (
        lax.dot_general(x_ref[...], w_ref[:, :D], dn,
                        preferred_element_type=jnp.float32)
        + lax.dot_general(r_ref[...], w_ref[:, D:], dn,
                          preferred_element_type=jnp.float32)
        + b_ref[...])


@jax.jit
def _tc_linear(x, r, W, b2):
    return pl.pallas_call(
        _mm_body,
        grid=(N // MT,),
        in_specs=[
            pl.BlockSpec((MT, D), lambda i: (i, 0)),
            pl.BlockSpec((MT, D), lambda i: (i, 0)),
            pl.BlockSpec((D, 2 * D), lambda i: (0, 0)),
            pl.BlockSpec((1, D), lambda i: (0, 0)),
        ],
        out_specs=pl.BlockSpec((MT, D), lambda i: (i, 0)),
        out_shape=jax.ShapeDtypeStruct((N, D), jnp.float32),
    )(x, r, W, b2)


def kernel(node_feats, edge_index, edge_feats, W, b):
    red = _sc_segment_sum(node_feats, edge_index[0], edge_index[1],
                          edge_feats.reshape(E))
    return _tc_linear(node_feats, red, W, b.reshape(1, D))


# speedup vs baseline: 8.8633x; 3.6461x over previous
import jax
import jax.numpy as jnp
from jax import lax
from jax.experimental import pallas as pl
from jax.experimental.pallas import tpu as pltpu
from jax.experimental.pallas import tpu_sc as plsc

N = 10000
E = 320000
D = 128
NC = 2
NS = 16
NW = NC * NS
L = 16

CH = 80
BLK = 25
NBLK = 5
EBLOCKS = NW * NBLK
CHKR = 80
NZCHK = N // CHKR


def _sc_body(x_hbm, src_hbm, dst_hbm, ef_hbm, out0_hbm, out1_hbm,
             acc_s, src_v, dst_v, ef_v, rows_a, rows_b, sem_a, sem_b):
    c = lax.axis_index("c")
    s = lax.axis_index("s")

    def _zero_row(i, carry):
        for k in range(D // L):
            rows_a[i, pl.ds(k * L, L)] = jnp.zeros((L,), jnp.float32)
        return carry
    lax.fori_loop(0, CHKR, _zero_row, None)
    for k in range(8):
        t = s + k * NS

        @pl.when(t < NZCHK)
        def _():
            pltpu.sync_copy(rows_a, acc_s.at[pl.ds(t * CHKR, CHKR)])
    plsc.subcore_barrier()

    w = c * NS + s
    gdn = lax.GatherDimensionNumbers(
        offset_dims=(), collapsed_slice_dims=(0,), start_index_map=(0,))

    def _process(j, rows_v):
        for g in range(CH // L):
            efg = ef_v[j, pl.ds(g * L, L)]
            for jl in range(L):
                efb = lax.gather(
                    efg, jnp.full((L, 1), jl, jnp.int32), gdn,
                    slice_sizes=(1,),
                    mode=lax.GatherScatterMode.PROMISE_IN_BOUNDS)
                jj = g * L + jl
                for k in range(D // L):
                    rows_v[jj, pl.ds(k * L, L)] = (
                        rows_v[jj, pl.ds(k * L, L)] * efb)
        pltpu.sync_copy(rows_v, acc_s.at[dst_v.at[j]], add=True)

    def _block(ob, carry):
        blk = w * NBLK + ob
        pltpu.sync_copy(src_hbm.at[blk], src_v)
        pltpu.sync_copy(dst_hbm.at[blk], dst_v)
        pltpu.sync_copy(ef_hbm.at[blk], ef_v)

        pltpu.async_copy(x_hbm.at[src_v.at[0]], rows_a, sem_a)

        def _pair(p, carry2):
            j = 2 * p
            pltpu.make_async_copy(x_hbm.at[src_v.at[j]], rows_a, sem_a).wait()
            pltpu.async_copy(x_hbm.at[src_v.at[j + 1]], rows_b, sem_b)
            _process(j, rows_a)
            pltpu.make_async_copy(
                x_hbm.at[src_v.at[j + 1]], rows_b, sem_b).wait()

            @pl.when(j + 2 < BLK)
            def _():
                pltpu.async_copy(x_hbm.at[src_v.at[j + 2]], rows_a, sem_a)
            _process(j + 1, rows_b)
            return carry2
        lax.fori_loop(0, BLK // 2, _pair, None)

        pltpu.make_async_copy(x_hbm.at[src_v.at[BLK - 1]], rows_a, sem_a).wait()
        _process(BLK - 1, rows_a)
        return carry
    lax.fori_loop(0, NBLK, _block, None)
    plsc.subcore_barrier()

    for k in range(8):
        t = s + k * NS

        @pl.when(t < NZCHK)
        def _():
            row = t * CHKR

            @pl.when(c == 0)
            def _():
                pltpu.sync_copy(acc_s.at[pl.ds(row, CHKR)],
                                out0_hbm.at[pl.ds(row, CHKR)])

            @pl.when(c == 1)
            def _():
                pltpu.sync_copy(acc_s.at[pl.ds(row, CHKR)],
                                out1_hbm.at[pl.ds(row, CHKR)])


@jax.jit
def _sc_segment_sum(x, src3, dst3, ef3):
    mesh = plsc.VectorSubcoreMesh(core_axis_name="c", subcore_axis_name="s")
    return pl.kernel(
        _sc_body,
        out_type=(jax.ShapeDtypeStruct((N, D), jnp.float32),
                  jax.ShapeDtypeStruct((N, D), jnp.float32)),
        mesh=mesh,
        scratch_types=[
            pltpu.VMEM_SHARED((N, D), jnp.float32),
            pltpu.VMEM((BLK, CH), jnp.int32),
            pltpu.VMEM((BLK, CH), jnp.int32),
            pltpu.VMEM((BLK, CH), jnp.float32),
            pltpu.VMEM((CH, D), jnp.float32),
            pltpu.VMEM((CH, D), jnp.float32),
            pltpu.SemaphoreType.DMA,
            pltpu.SemaphoreType.DMA,
        ],
    )(x, src3, dst3, ef3)


MT = 2000


def _mm_body(x_ref, r0_ref, r1_ref, w_ref, b_ref, o_ref):
    dn = (((1,), (1,)), ((), ()))
    o_ref[...] = (
        lax.dot_general(x_ref[...], w_ref[:, :D], dn,
                        preferred_element_type=jnp.float32)
        + lax.dot_general(r0_ref[...] + r1_ref[...], w_ref[:, D:], dn,
                          preferred_element_type=jnp.float32)
        + b_ref[...])


@jax.jit
def _tc_linear(x, r0, r1, W, b2):
    return pl.pallas_call(
        _mm_body,
        grid=(N // MT,),
        in_specs=[
            pl.BlockSpec((MT, D), lambda i: (i, 0)),
            pl.BlockSpec((MT, D), lambda i: (i, 0)),
            pl.BlockSpec((MT, D), lambda i: (i, 0)),
            pl.BlockSpec((D, 2 * D), lambda i: (0, 0)),
            pl.BlockSpec((1, D), lambda i: (0, 0)),
        ],
        out_specs=pl.BlockSpec((MT, D), lambda i: (i, 0)),
        out_shape=jax.ShapeDtypeStruct((N, D), jnp.float32),
    )(x, r0, r1, W, b2)


def kernel(node_feats, edge_index, edge_feats, W, b):
    src3 = edge_index[0].reshape(EBLOCKS, BLK, CH)
    dst3 = edge_index[1].reshape(EBLOCKS, BLK, CH)
    ef3 = edge_feats.reshape(EBLOCKS, BLK, CH)
    r0, r1 = _sc_segment_sum(node_feats, src3, dst3, ef3)
    return _tc_linear(node_feats, r0, r1, W, b.reshape(1, D))
